# baseline (speedup 1.0000x reference)
import functools

import jax
import jax.numpy as jnp
from jax import lax
from jax.experimental import pallas as pl
from jax.experimental.pallas import tpu as pltpu
from jax.experimental.pallas import tpu_sc as plsc

R = 128
N = 32768
K = (N - 1) // 2
L = 16
NW = 32
RPW = R // NW
UC = 8
U = 8
NCOUNT = 5
SEG = N // L
REGS = SEG + 1
BLEN = (L - 1) * REGS + SEG + L
HI0 = 0x3F7FFFFF
MAXIT = 64

_mesh = plsc.VectorSubcoreMesh(core_axis_name="c", subcore_axis_name="s")


def _xsum(v):
    return plsc.cumsum(v)[L - 1]


@functools.partial(
    pl.kernel,
    mesh=_mesh,
    out_type=jax.ShapeDtypeStruct((NW, L), jnp.float32),
    compiler_params=pltpu.CompilerParams(needs_layout_passes=False),
    scratch_types=[
        pltpu.VMEM((BLEN,), jnp.float32),
        pltpu.VMEM((BLEN,), jnp.float32),
        pltpu.VMEM((BLEN,), jnp.float32),
        pltpu.VMEM((2 * L,), jnp.float32),
        pltpu.VMEM((L,), jnp.float32),
    ],
)
def _soft_majority_sc(x_hbm, out_hbm, buf0, buf1, buf2, tiny_v, res_v):
    wid = lax.axis_index("s") * 2 + lax.axis_index("c")
    lane = lax.iota(jnp.int32, L)
    zero_i = jnp.zeros((L,), jnp.int32)
    one_i = jnp.ones((L,), jnp.int32)
    base_v = lane * REGS

    def mk_probe(lo_v, hi_v):
        lo_f = lax.bitcast_convert_type(lo_v, jnp.float32)
        hi_f = lax.bitcast_convert_type(hi_v, jnp.float32)
        vmid = 0.5 * (lo_f + hi_f)
        pmid = lax.bitcast_convert_type(vmid, jnp.int32)
        probe_pat = jnp.minimum(jnp.maximum(pmid, lo_v), hi_v - 1)
        return probe_pat, lax.bitcast_convert_type(probe_pat, jnp.float32)

    def count_pass(src, probe_f, with_sum):
        def body(i, carry):
            if with_sum:
                cacc, acc = carry
            else:
                cacc = carry
            b = i * (L * UC)
            for u in range(UC):
                v = src[pl.ds(b + u * L, L)]
                cacc = cacc + plsc.all_reduce_population_count(v <= probe_f)
                if with_sum:
                    acc = acc + v
            return (cacc, acc) if with_sum else cacc

        init = (zero_i, jnp.zeros((L,), jnp.float32)) if with_sum else zero_i
        out = lax.fori_loop(0, SEG // UC, body, init)
        if with_sum:
            return out[0][0], out[1]
        return out[0], None

    def extract(src, dst, lo_f, hi_f):
        def body(i, off):
            b = i * (L * U)
            for u in range(U):
                v = src[pl.ds(b + u * L, L)]
                m = jnp.logical_and(v >= lo_f, v <= hi_f)
                sk, _, _ = plsc.sort_key_val(v, v, mask=m)
                dst[pl.ds(off, L)] = sk
                off = off + plsc.all_reduce_population_count(m)[0]
            return off

        return lax.fori_loop(0, SEG // U, body, jnp.int32(0))

    def splitn(src, d_lo, d_hi, s_vec, base_cur, probe_f):
        trip = plsc.cummax(s_vec)[L - 1]

        def body(i, carry):
            offL, offH = carry
            for u in range(U):
                iu = i * U + u
                idx = base_cur + iu
                v = plsc.load_gather(src, [idx])
                valid = s_vec > iu
                le0 = v <= probe_f
                m = jnp.logical_and(le0, valid)
                mh = jnp.logical_and(jnp.logical_not(le0), valid)
                plsc.store_scatter(d_lo, [offL], v, mask=m)
                plsc.store_scatter(d_hi, [offH], v, mask=mh)
                offL = offL + jnp.where(m, one_i, zero_i)
                offH = offH + jnp.where(mh, one_i, zero_i)
            return (offL, offH)

        nit = (trip + (U - 1)) // U
        offL, offH = lax.fori_loop(0, nit, body, (base_v, base_v))
        return offL - base_v, offH - base_v

    def split_from(src_id, s_vec, base_cur, probe_f):
        return lax.cond(
            src_id == 0,
            lambda: splitn(buf0, buf1, buf2, s_vec, base_cur, probe_f),
            lambda: lax.cond(
                src_id == 1,
                lambda: splitn(buf1, buf0, buf2, s_vec, base_cur, probe_f),
                lambda: splitn(buf2, buf0, buf1, s_vec, base_cur, probe_f)))

    def collect(src_id, s_vec, base_cur, size_s, k_s):
        def coll(src):
            off = jnp.int32(0)
            for i in range(L):
                v = plsc.load_gather(src, [base_cur + i])
                valid = s_vec > i
                plsc.store_compressed(tiny_v.at[pl.ds(off, L)], v,
                                      mask=valid)
                off = off + plsc.all_reduce_population_count(valid)[0]
            w = tiny_v[pl.ds(0, L)]
            valid2 = lane < jnp.broadcast_to(size_s, (L,))
            skeys, _, _ = plsc.sort_key_val(w, w, mask=valid2)
            sel = jnp.where(lane == jnp.broadcast_to(k_s, (L,)), skeys, 0.0)
            return _xsum(sel)
        return lax.cond(
            src_id == 0, lambda: coll(buf0),
            lambda: lax.cond(src_id == 1, lambda: coll(buf1),
                             lambda: coll(buf2)))

    def process_row(jj, res):
        row = wid * RPW + jj
        pltpu.sync_copy(x_hbm.at[row], buf0.at[pl.ds(0, N)])

        lo_v = zero_i
        hi_v = jnp.full((L,), HI0, jnp.int32)
        kk = jnp.int32(K)
        below = jnp.int32(0)
        mean = jnp.float32(0.0)
        for p in range(NCOUNT):
            if p == 0:
                probe_pat = jnp.full((L,), 0x3F000000, jnp.int32)
                probe_f = jnp.full((L,), 0.5, jnp.float32)
                c_le, acc = count_pass(buf0, probe_f, True)
                mean = _xsum(acc) * (1.0 / N)
            else:
                probe_pat, probe_f = mk_probe(lo_v, hi_v)
                c_le, _ = count_pass(buf0, probe_f, False)
            cin = c_le - below
            go = kk < cin
            go_v = jnp.broadcast_to(go, (L,))
            hi_v = jnp.where(go_v, probe_pat, hi_v)
            lo_v = jnp.where(go_v, lo_v, probe_pat + 1)
            kk = jnp.where(go, kk, kk - cin)
            below = jnp.where(go, below, c_le)

        size1 = extract(buf0, buf1,
                        lax.bitcast_convert_type(lo_v, jnp.float32),
                        lax.bitcast_convert_type(hi_v, jnp.float32))
        seg_len = (size1 + (L - 1)) // L
        base1 = lane * jnp.broadcast_to(seg_len, (L,))
        s1 = jnp.clip(jnp.broadcast_to(size1, (L,)) - base1, 0,
                      jnp.broadcast_to(seg_len, (L,)))

        def w_cond(st):
            src_id, base_cur, s_vec, size_s, k_s, it, lo_v, hi_v = st
            return (size_s > L) & (lo_v[0] < hi_v[0]) & (it < MAXIT)

        def w_body(st):
            src_id, base_cur, s_vec, size_s, k_s, it, lo_v, hi_v = st
            probe_pat, probe_f = mk_probe(lo_v, hi_v)
            sL2, sH2 = split_from(src_id, s_vec, base_cur, probe_f)
            cL2 = _xsum(sL2)
            go2 = k_s < cL2
            go_v = jnp.broadcast_to(go2, (L,))
            k2 = jnp.where(go2, k_s, k_s - cL2)
            size2 = jnp.where(go2, cL2, size_s - cL2)
            s2 = jnp.where(go_v, sL2, sH2)
            hi2 = jnp.where(go_v, probe_pat, hi_v)
            lo2 = jnp.where(go_v, lo_v, probe_pat + 1)
            lodest = jnp.where(src_id == 0, 1, 0)
            hidest = jnp.where(src_id == 2, 1, 2)
            src2 = jnp.where(go2, lodest, hidest)
            return (src2, base_v, s2, size2, k2, it + 1, lo2, hi2)

        src_id, base_cur, s_vec, size_s, k_s, _, lo_v, _ = lax.while_loop(
            w_cond, w_body,
            (jnp.int32(1), base1, s1, size1, kk, jnp.int32(0), lo_v, hi_v))

        m_sorted = lax.cond(
            size_s <= L,
            lambda: collect(src_id, s_vec, base_cur, size_s, k_s),
            lambda: jnp.float32(0.0))
        m_bit = jnp.where(size_s <= L, m_sorted,
                          lax.bitcast_convert_type(lo_v, jnp.float32)[0])

        margin = jnp.abs(m_bit - 0.5)
        md = mean * margin
        rep = jnp.where(m_bit > 0.5, 0.5 + md, m_bit + md)
        return jnp.where(lane == jnp.broadcast_to(jj, (L,)),
                         jnp.broadcast_to(rep, (L,)), res)

    res = lax.fori_loop(0, RPW, process_row, jnp.zeros((L,), jnp.float32))
    res_v[...] = res
    pltpu.sync_copy(res_v, out_hbm.at[wid])


def kernel(x):
    padded = _soft_majority_sc(x)
    return padded[:, :RPW].reshape(R)


# speedup vs baseline: 1.3456x; 1.3456x over previous
import functools

import jax
import jax.numpy as jnp
from jax import lax
from jax.experimental import pallas as pl
from jax.experimental.pallas import tpu as pltpu
from jax.experimental.pallas import tpu_sc as plsc

R = 128
N = 32768
K = (N - 1) // 2
L = 16
NW = 32
RPW = R // NW
UC = 8
U = 4
NCOUNT = 5
SEG = N // L
REGS = SEG + 1
BLEN = (L - 1) * REGS + SEG + L
HI0 = 0x3F7FFFFF
MAXIT = 64

_mesh = plsc.VectorSubcoreMesh(core_axis_name="c", subcore_axis_name="s")


def _xsum(v):
    return plsc.cumsum(v)[L - 1]


@functools.partial(
    pl.kernel,
    mesh=_mesh,
    out_type=jax.ShapeDtypeStruct((NW, L), jnp.float32),
    compiler_params=pltpu.CompilerParams(needs_layout_passes=False),
    scratch_types=[
        pltpu.VMEM((BLEN,), jnp.float32),
        pltpu.VMEM((BLEN,), jnp.float32),
        pltpu.VMEM((BLEN,), jnp.float32),
        pltpu.VMEM((2 * L,), jnp.float32),
        pltpu.VMEM((L,), jnp.float32),
    ],
)
def _soft_majority_sc(x_hbm, out_hbm, buf0, buf1, buf2, tiny_v, res_v):
    wid = lax.axis_index("s") * 2 + lax.axis_index("c")
    lane = lax.iota(jnp.int32, L)
    zero_i = jnp.zeros((L,), jnp.int32)
    one_i = jnp.ones((L,), jnp.int32)
    base_v = lane * REGS

    def mk_probe(lo_v, hi_v):
        lo_f = lax.bitcast_convert_type(lo_v, jnp.float32)
        hi_f = lax.bitcast_convert_type(hi_v, jnp.float32)
        vmid = 0.5 * (lo_f + hi_f)
        pmid = lax.bitcast_convert_type(vmid, jnp.int32)
        probe_pat = jnp.minimum(jnp.maximum(pmid, lo_v), hi_v - 1)
        return probe_pat, lax.bitcast_convert_type(probe_pat, jnp.float32)

    def count_pass(src, probe_f, with_sum):
        def body(i, carry):
            if with_sum:
                cacc, acc = carry
            else:
                cacc = carry
            b = i * (L * UC)
            for u in range(UC):
                v = src[pl.ds(b + u * L, L)]
                cacc = cacc + plsc.all_reduce_population_count(v <= probe_f)
                if with_sum:
                    acc = acc + v
            return (cacc, acc) if with_sum else cacc

        init = (zero_i, jnp.zeros((L,), jnp.float32)) if with_sum else zero_i
        out = lax.fori_loop(0, SEG // UC, body, init)
        if with_sum:
            return out[0][0], out[1]
        return out[0], None

    def extract(src, dst, lo_f, hi_f):
        def body(i, off):
            b = i * (L * U)
            for u in range(U):
                v = src[pl.ds(b + u * L, L)]
                m = jnp.logical_and(v >= lo_f, v <= hi_f)
                plsc.store_scatter(dst, [off], v, mask=m)
                off = off + jnp.where(m, one_i, zero_i)
            return off

        off = lax.fori_loop(0, SEG // U, body, base_v)
        return off - base_v

    def splitn(src, d_lo, d_hi, s_vec, probe_f):
        trip = plsc.cummax(s_vec)[L - 1]

        def body(i, carry):
            offL, offH = carry
            for u in range(U):
                iu = i * U + u
                idx = base_v + iu
                v = plsc.load_gather(src, [idx])
                valid = s_vec > iu
                le0 = v <= probe_f
                m = jnp.logical_and(le0, valid)
                mh = jnp.logical_and(jnp.logical_not(le0), valid)
                plsc.store_scatter(d_lo, [offL], v, mask=m)
                plsc.store_scatter(d_hi, [offH], v, mask=mh)
                offL = offL + jnp.where(m, one_i, zero_i)
                offH = offH + jnp.where(mh, one_i, zero_i)
            return (offL, offH)

        nit = (trip + (U - 1)) // U
        offL, offH = lax.fori_loop(0, nit, body, (base_v, base_v))
        return offL - base_v, offH - base_v

    def split_from(src_id, s_vec, probe_f):
        return lax.cond(
            src_id == 0,
            lambda: splitn(buf0, buf1, buf2, s_vec, probe_f),
            lambda: lax.cond(
                src_id == 1,
                lambda: splitn(buf1, buf0, buf2, s_vec, probe_f),
                lambda: splitn(buf2, buf0, buf1, s_vec, probe_f)))

    def collect(src_id, s_vec, size_s, k_s):
        def coll(src):
            off = jnp.int32(0)
            for i in range(L):
                v = plsc.load_gather(src, [base_v + i])
                valid = s_vec > i
                plsc.store_compressed(tiny_v.at[pl.ds(off, L)], v,
                                      mask=valid)
                off = off + plsc.all_reduce_population_count(valid)[0]
            w = tiny_v[pl.ds(0, L)]
            valid2 = lane < jnp.broadcast_to(size_s, (L,))
            skeys, _, _ = plsc.sort_key_val(w, w, mask=valid2)
            sel = jnp.where(lane == jnp.broadcast_to(k_s, (L,)), skeys, 0.0)
            return _xsum(sel)
        return lax.cond(
            src_id == 0, lambda: coll(buf0),
            lambda: lax.cond(src_id == 1, lambda: coll(buf1),
                             lambda: coll(buf2)))

    def process_row(jj, res):
        row = wid * RPW + jj
        pltpu.sync_copy(x_hbm.at[row], buf0.at[pl.ds(0, N)])

        lo_v = zero_i
        hi_v = jnp.full((L,), HI0, jnp.int32)
        kk = jnp.int32(K)
        below = jnp.int32(0)
        mean = jnp.float32(0.0)
        for p in range(NCOUNT):
            if p == 0:
                probe_pat = jnp.full((L,), 0x3F000000, jnp.int32)
                probe_f = jnp.full((L,), 0.5, jnp.float32)
                c_le, acc = count_pass(buf0, probe_f, True)
                mean = _xsum(acc) * (1.0 / N)
            else:
                probe_pat, probe_f = mk_probe(lo_v, hi_v)
                c_le, _ = count_pass(buf0, probe_f, False)
            cin = c_le - below
            go = kk < cin
            go_v = jnp.broadcast_to(go, (L,))
            hi_v = jnp.where(go_v, probe_pat, hi_v)
            lo_v = jnp.where(go_v, lo_v, probe_pat + 1)
            kk = jnp.where(go, kk, kk - cin)
            below = jnp.where(go, below, c_le)

        s1 = extract(buf0, buf1,
                     lax.bitcast_convert_type(lo_v, jnp.float32),
                     lax.bitcast_convert_type(hi_v, jnp.float32))
        size1 = _xsum(s1)

        def w_cond(st):
            src_id, s_vec, size_s, k_s, it, lo_v, hi_v = st
            return (size_s > L) & (lo_v[0] < hi_v[0]) & (it < MAXIT)

        def w_body(st):
            src_id, s_vec, size_s, k_s, it, lo_v, hi_v = st
            probe_pat, probe_f = mk_probe(lo_v, hi_v)
            sL2, sH2 = split_from(src_id, s_vec, probe_f)
            cL2 = _xsum(sL2)
            go2 = k_s < cL2
            go_v = jnp.broadcast_to(go2, (L,))
            k2 = jnp.where(go2, k_s, k_s - cL2)
            size2 = jnp.where(go2, cL2, size_s - cL2)
            s2 = jnp.where(go_v, sL2, sH2)
            hi2 = jnp.where(go_v, probe_pat, hi_v)
            lo2 = jnp.where(go_v, lo_v, probe_pat + 1)
            lodest = jnp.where(src_id == 0, 1, 0)
            hidest = jnp.where(src_id == 2, 1, 2)
            src2 = jnp.where(go2, lodest, hidest)
            return (src2, s2, size2, k2, it + 1, lo2, hi2)

        src_id, s_vec, size_s, k_s, _, lo_v, _ = lax.while_loop(
            w_cond, w_body,
            (jnp.int32(1), s1, size1, kk, jnp.int32(0), lo_v, hi_v))

        m_sorted = lax.cond(
            size_s <= L,
            lambda: collect(src_id, s_vec, size_s, k_s),
            lambda: jnp.float32(0.0))
        m_bit = jnp.where(size_s <= L, m_sorted,
                          lax.bitcast_convert_type(lo_v, jnp.float32)[0])

        margin = jnp.abs(m_bit - 0.5)
        md = mean * margin
        rep = jnp.where(m_bit > 0.5, 0.5 + md, m_bit + md)
        return jnp.where(lane == jnp.broadcast_to(jj, (L,)),
                         jnp.broadcast_to(rep, (L,)), res)

    res = lax.fori_loop(0, RPW, process_row, jnp.zeros((L,), jnp.float32))
    res_v[...] = res
    pltpu.sync_copy(res_v, out_hbm.at[wid])


def kernel(x):
    padded = _soft_majority_sc(x)
    return padded[:, :RPW].reshape(R)


# speedup vs baseline: 1.3563x; 1.0079x over previous
import functools

import jax
import jax.numpy as jnp
from jax import lax
from jax.experimental import pallas as pl
from jax.experimental.pallas import tpu as pltpu
from jax.experimental.pallas import tpu_sc as plsc

R = 128
N = 32768
K = (N - 1) // 2
L = 16
NW = 32
RPW = R // NW
UC = 8
U = 8
NCOUNT = 5
SEG = N // L
REGS = SEG + 1
BLEN = (L - 1) * REGS + SEG + L
HI0 = 0x3F7FFFFF
MAXIT = 64

_mesh = plsc.VectorSubcoreMesh(core_axis_name="c", subcore_axis_name="s")


def _xsum(v):
    return plsc.cumsum(v)[L - 1]


@functools.partial(
    pl.kernel,
    mesh=_mesh,
    out_type=jax.ShapeDtypeStruct((NW, L), jnp.float32),
    compiler_params=pltpu.CompilerParams(needs_layout_passes=False),
    scratch_types=[
        pltpu.VMEM((BLEN,), jnp.float32),
        pltpu.VMEM((BLEN,), jnp.float32),
        pltpu.VMEM((BLEN,), jnp.float32),
        pltpu.VMEM((2 * L,), jnp.float32),
        pltpu.VMEM((L,), jnp.float32),
    ],
)
def _soft_majority_sc(x_hbm, out_hbm, buf0, buf1, buf2, tiny_v, res_v):
    wid = lax.axis_index("s") * 2 + lax.axis_index("c")
    lane = lax.iota(jnp.int32, L)
    zero_i = jnp.zeros((L,), jnp.int32)
    one_i = jnp.ones((L,), jnp.int32)
    base_v = lane * REGS

    def mk_probe(lo_v, hi_v):
        lo_f = lax.bitcast_convert_type(lo_v, jnp.float32)
        hi_f = lax.bitcast_convert_type(hi_v, jnp.float32)
        vmid = 0.5 * (lo_f + hi_f)
        pmid = lax.bitcast_convert_type(vmid, jnp.int32)
        probe_pat = jnp.minimum(jnp.maximum(pmid, lo_v), hi_v - 1)
        return probe_pat, lax.bitcast_convert_type(probe_pat, jnp.float32)

    def count_pass(src, probe_f, with_sum):
        def body(i, carry):
            if with_sum:
                cacc, acc = carry
            else:
                cacc = carry
            b = i * (L * UC)
            for u in range(UC):
                v = src[pl.ds(b + u * L, L)]
                cacc = cacc + plsc.all_reduce_population_count(v <= probe_f)
                if with_sum:
                    acc = acc + v
            return (cacc, acc) if with_sum else cacc

        init = (zero_i, jnp.zeros((L,), jnp.float32)) if with_sum else zero_i
        out = lax.fori_loop(0, SEG // UC, body, init)
        if with_sum:
            return out[0][0], out[1]
        return out[0], None

    def extract(src, dst, lo_f, hi_f):
        def body(i, off):
            b = i * (L * U)
            for u in range(U):
                v = src[pl.ds(b + u * L, L)]
                m = jnp.logical_and(v >= lo_f, v <= hi_f)
                plsc.store_scatter(dst, [off], v, mask=m)
                off = off + jnp.where(m, one_i, zero_i)
            return off

        off = lax.fori_loop(0, SEG // U, body, base_v)
        return off - base_v

    def splitn(src, d_lo, d_hi, s_vec, probe_f):
        trip = plsc.cummax(s_vec)[L - 1]

        def body(i, carry):
            offL, offH = carry
            for u in range(U):
                iu = i * U + u
                idx = base_v + iu
                v = plsc.load_gather(src, [idx])
                valid = s_vec > iu
                le0 = v <= probe_f
                m = jnp.logical_and(le0, valid)
                mh = jnp.logical_and(jnp.logical_not(le0), valid)
                plsc.store_scatter(d_lo, [offL], v, mask=m)
                plsc.store_scatter(d_hi, [offH], v, mask=mh)
                offL = offL + jnp.where(m, one_i, zero_i)
                offH = offH + jnp.where(mh, one_i, zero_i)
            return (offL, offH)

        nit = (trip + (U - 1)) // U
        offL, offH = lax.fori_loop(0, nit, body, (base_v, base_v))
        return offL - base_v, offH - base_v

    def split_from(src_id, s_vec, probe_f):
        return lax.cond(
            src_id == 0,
            lambda: splitn(buf0, buf1, buf2, s_vec, probe_f),
            lambda: lax.cond(
                src_id == 1,
                lambda: splitn(buf1, buf0, buf2, s_vec, probe_f),
                lambda: splitn(buf2, buf0, buf1, s_vec, probe_f)))

    def collect(src_id, s_vec, size_s, k_s):
        def coll(src):
            off = jnp.int32(0)
            for i in range(L):
                v = plsc.load_gather(src, [base_v + i])
                valid = s_vec > i
                plsc.store_compressed(tiny_v.at[pl.ds(off, L)], v,
                                      mask=valid)
                off = off + plsc.all_reduce_population_count(valid)[0]
            w = tiny_v[pl.ds(0, L)]
            valid2 = lane < jnp.broadcast_to(size_s, (L,))
            skeys, _, _ = plsc.sort_key_val(w, w, mask=valid2)
            sel = jnp.where(lane == jnp.broadcast_to(k_s, (L,)), skeys, 0.0)
            return _xsum(sel)
        return lax.cond(
            src_id == 0, lambda: coll(buf0),
            lambda: lax.cond(src_id == 1, lambda: coll(buf1),
                             lambda: coll(buf2)))

    def process_row(jj, res):
        row = wid * RPW + jj
        pltpu.sync_copy(x_hbm.at[row], buf0.at[pl.ds(0, N)])

        lo_v = zero_i
        hi_v = jnp.full((L,), HI0, jnp.int32)
        kk = jnp.int32(K)
        below = jnp.int32(0)
        mean = jnp.float32(0.0)
        for p in range(NCOUNT):
            if p == 0:
                probe_pat = jnp.full((L,), 0x3F000000, jnp.int32)
                probe_f = jnp.full((L,), 0.5, jnp.float32)
                c_le, acc = count_pass(buf0, probe_f, True)
                mean = _xsum(acc) * (1.0 / N)
            else:
                probe_pat, probe_f = mk_probe(lo_v, hi_v)
                c_le, _ = count_pass(buf0, probe_f, False)
            cin = c_le - below
            go = kk < cin
            go_v = jnp.broadcast_to(go, (L,))
            hi_v = jnp.where(go_v, probe_pat, hi_v)
            lo_v = jnp.where(go_v, lo_v, probe_pat + 1)
            kk = jnp.where(go, kk, kk - cin)
            below = jnp.where(go, below, c_le)

        s1 = extract(buf0, buf1,
                     lax.bitcast_convert_type(lo_v, jnp.float32),
                     lax.bitcast_convert_type(hi_v, jnp.float32))
        size1 = _xsum(s1)

        def w_cond(st):
            src_id, s_vec, size_s, k_s, it, lo_v, hi_v = st
            return (size_s > L) & (lo_v[0] < hi_v[0]) & (it < MAXIT)

        def w_body(st):
            src_id, s_vec, size_s, k_s, it, lo_v, hi_v = st
            probe_pat, probe_f = mk_probe(lo_v, hi_v)
            sL2, sH2 = split_from(src_id, s_vec, probe_f)
            cL2 = _xsum(sL2)
            go2 = k_s < cL2
            go_v = jnp.broadcast_to(go2, (L,))
            k2 = jnp.where(go2, k_s, k_s - cL2)
            size2 = jnp.where(go2, cL2, size_s - cL2)
            s2 = jnp.where(go_v, sL2, sH2)
            hi2 = jnp.where(go_v, probe_pat, hi_v)
            lo2 = jnp.where(go_v, lo_v, probe_pat + 1)
            lodest = jnp.where(src_id == 0, 1, 0)
            hidest = jnp.where(src_id == 2, 1, 2)
            src2 = jnp.where(go2, lodest, hidest)
            return (src2, s2, size2, k2, it + 1, lo2, hi2)

        src_id, s_vec, size_s, k_s, _, lo_v, _ = lax.while_loop(
            w_cond, w_body,
            (jnp.int32(1), s1, size1, kk, jnp.int32(0), lo_v, hi_v))

        m_sorted = lax.cond(
            size_s <= L,
            lambda: collect(src_id, s_vec, size_s, k_s),
            lambda: jnp.float32(0.0))
        m_bit = jnp.where(size_s <= L, m_sorted,
                          lax.bitcast_convert_type(lo_v, jnp.float32)[0])

        margin = jnp.abs(m_bit - 0.5)
        md = mean * margin
        rep = jnp.where(m_bit > 0.5, 0.5 + md, m_bit + md)
        return jnp.where(lane == jnp.broadcast_to(jj, (L,)),
                         jnp.broadcast_to(rep, (L,)), res)

    res = lax.fori_loop(0, RPW, process_row, jnp.zeros((L,), jnp.float32))
    res_v[...] = res
    pltpu.sync_copy(res_v, out_hbm.at[wid])


def kernel(x):
    padded = _soft_majority_sc(x)
    return padded[:, :RPW].reshape(R)
